# Initial kernel scaffold; baseline (speedup 1.0000x reference)
import jax
import jax.numpy as jnp
from jax.experimental import pallas as pl


def kernel(fea, xyz, batch, params1, params2, params3):
    raise NotImplementedError("write your pallas kernel here")


# speedup vs baseline: 7.2600x; 7.2600x over previous
import functools

import jax
import jax.numpy as jnp
from jax import lax
from jax.experimental import pallas as pl
from jax.experimental.pallas import tpu as pltpu
from jax.experimental.pallas import tpu_sc as plsc

_K = 32
_F32 = jnp.float32
_I32 = jnp.int32


def _tree3(a, b, c):
    return (a + c) + b


def _fps_body(n, n_s, R, C, xs_ref, ys_ref, zs_ref, idx_ref, poss_ref, d_ref):
    lin = (lax.broadcasted_iota(_I32, (R, C), 0) * C
           + lax.broadcasted_iota(_I32, (R, C), 1))

    def point_of(i_sel):
        sel = lin == i_sel
        px = jnp.max(jnp.where(sel, xs_ref[...], -jnp.inf))
        py = jnp.max(jnp.where(sel, ys_ref[...], -jnp.inf))
        pz = jnp.max(jnp.where(sel, zs_ref[...], -jnp.inf))
        return px, py, pz

    d_ref[...] = jnp.full((R, C), jnp.inf, _F32)
    idx_ref[0:1, :] = jnp.zeros((1, 1), _I32)

    def body(i, last):
        px, py, pz = point_of(last)
        poss_ref[pl.ds(i - 1, 1), :] = jnp.reshape(
            jnp.stack([px, py, pz]), (1, 3))
        dx = xs_ref[...] - px
        dy = ys_ref[...] - py
        dz = zs_ref[...] - pz
        dnew = _tree3(dx * dx, dy * dy, dz * dz)
        dm = jnp.minimum(d_ref[...], dnew)
        d_ref[...] = dm
        m = jnp.max(dm)
        j = jnp.min(jnp.where(dm == m, lin, n))
        idx_ref[pl.ds(i, 1), :] = jnp.broadcast_to(j, (1, 1))
        return j

    last = lax.fori_loop(1, n_s, body, jnp.int32(0))
    px, py, pz = point_of(last)
    poss_ref[pl.ds(n_s - 1, 1), :] = jnp.reshape(
        jnp.stack([px, py, pz]), (1, 3))


def _fps(pos, n_s):
    n = pos.shape[0]
    C = 128
    R = n // C
    xs = pos[:, 0].reshape(R, C)
    ys = pos[:, 1].reshape(R, C)
    zs = pos[:, 2].reshape(R, C)
    idx, poss = pl.pallas_call(
        functools.partial(_fps_body, n, n_s, R, C),
        out_shape=(jax.ShapeDtypeStruct((n_s, 1), _I32),
                   jax.ShapeDtypeStruct((n_s, 3), _F32)),
        scratch_shapes=[pltpu.VMEM((R, C), _F32)],
    )(xs, ys, zs)
    return idx[:, 0], poss


def _knn_body(n, TQ, CH, xa_ref, ya_ref, za_ref, qx_ref, qy_ref, qz_ref,
              col_ref, d_ref):
    nch = n // CH
    qx = qx_ref[...]
    qy = qy_ref[...]
    qz = qz_ref[...]
    q2 = _tree3(qx * qx, qy * qy, qz * qz)

    def bf(v):
        return v.astype(jnp.bfloat16).astype(_F32)

    qxb, qyb, qzb = bf(qx), bf(qy), bf(qz)

    for c in range(nch):
        sl = pl.ds(c * CH, CH)
        ax = xa_ref[:, sl]
        ay = ya_ref[:, sl]
        az = za_ref[:, sl]
        a2 = _tree3(ax * ax, ay * ay, az * az)
        p0 = qxb * bf(ax)
        p1 = qyb * bf(ay)
        p2 = qzb * bf(az)
        s = p0 + p1
        bp = s - p0
        e = (p0 - (s - bp)) + (p1 - bp)
        s2 = s + p2
        bp2 = s2 - s
        e2 = (s - (s2 - bp2)) + (p2 - bp2)
        qdot = s2 + (e + e2)
        d_ref[:, sl] = (q2 + a2) - 2.0 * qdot

    col_iota = lax.broadcasted_iota(_I32, (TQ, _K), 1)
    cols = jnp.zeros((TQ, _K), _I32)
    for r in range(_K):
        m = jnp.full((TQ, 1), jnp.inf, _F32)
        for c in range(nch):
            blk = d_ref[:, pl.ds(c * CH, CH)]
            m = jnp.minimum(m, jnp.min(blk, axis=1, keepdims=True))
        j = jnp.full((TQ, 1), n, _I32)
        for c in range(nch):
            blk = d_ref[:, pl.ds(c * CH, CH)]
            lin_c = lax.broadcasted_iota(_I32, (TQ, CH), 1) + c * CH
            j = jnp.minimum(
                j, jnp.min(jnp.where(blk == m, lin_c, n), axis=1,
                           keepdims=True))
        cols = jnp.where(col_iota == r, jnp.broadcast_to(j, (TQ, _K)), cols)
        for c in range(nch):
            sl = pl.ds(c * CH, CH)
            blk = d_ref[:, sl]
            lin_c = lax.broadcasted_iota(_I32, (TQ, CH), 1) + c * CH
            d_ref[:, sl] = jnp.where(lin_c == j, jnp.inf, blk)
    col_ref[...] = cols


def _knn(pos_all, pos_q):
    n = pos_all.shape[0]
    n_q = pos_q.shape[0]
    TQ = 16
    CH = 2048 if n >= 2048 else n
    qspec = pl.BlockSpec((TQ, 1), lambda i: (i, 0))
    aspec = pl.BlockSpec((1, n), lambda i: (0, 0))
    col = pl.pallas_call(
        functools.partial(_knn_body, n, TQ, CH),
        grid=(n_q // TQ,),
        in_specs=[aspec, aspec, aspec, qspec, qspec, qspec],
        out_specs=pl.BlockSpec((TQ, _K), lambda i: (i, 0)),
        out_shape=jax.ShapeDtypeStruct((n_q, _K), _I32),
        scratch_shapes=[pltpu.VMEM((TQ, n), _F32)],
    )(pos_all[:, 0].reshape(1, n), pos_all[:, 1].reshape(1, n),
      pos_all[:, 2].reshape(1, n), pos_q[:, 0].reshape(n_q, 1),
      pos_q[:, 1].reshape(n_q, 1), pos_q[:, 2].reshape(n_q, 1))
    return col


def _sc_gather(table, idx_flat, CR):
    n, D = table.shape
    E = idx_flat.shape[0]
    NW = 32
    rpw = E // NW
    n_chunks = rpw // CR
    n_dma = CR // 128
    mesh = plsc.VectorSubcoreMesh(core_axis_name="c", subcore_axis_name="s")

    @functools.partial(
        pl.kernel,
        out_type=jax.ShapeDtypeStruct((E, D), _F32),
        mesh=mesh,
        scratch_types=[
            pltpu.VMEM((rpw,), _I32),
            pltpu.VMEM((CR, D), _F32),
            pltpu.SemaphoreType.DMA,
        ],
        compiler_params=pltpu.CompilerParams(use_tc_tiling_on_sc=False),
    )
    def gather_kernel(table_hbm, idx_hbm, out_hbm, idx_v, rows_v, sem):
        wid = lax.axis_index("s") * 2 + lax.axis_index("c")
        base = pl.multiple_of(wid * rpw, rpw)
        pltpu.sync_copy(idx_hbm.at[pl.ds(base, rpw)], idx_v)

        def chunk_body(ci, _):
            row0 = pl.multiple_of(base + ci * CR, CR)
            copies = []
            for j in range(n_dma):
                copies.append(pltpu.async_copy(
                    table_hbm.at[idx_v.at[pl.ds(ci * CR + j * 128, 128)]],
                    rows_v.at[pl.ds(j * 128, 128)], sem))
            for cp in copies:
                cp.wait()
            pltpu.sync_copy(rows_v, out_hbm.at[pl.ds(row0, CR)])
            return 0

        lax.fori_loop(0, n_chunks, chunk_body, 0)

    return gather_kernel(table, idx_flat)


def _bf(v):
    return v.astype(jnp.bfloat16)


def _l1_body(T, F, e_ref, q_ref, w_ref, b_ref, z_ref, s_ref, ss_ref):
    i = pl.program_id(0)
    xpart = e_ref[:, 0:F]
    ppart = (e_ref[:, F:F + 3].reshape(T // _K, _K, 3)
             - q_ref[...][:, None, :]).reshape(T, 3)
    x6 = jnp.concatenate([xpart, ppart], axis=1)
    z = jnp.dot(_bf(x6), w_ref[...], preferred_element_type=_F32)
    z = z + b_ref[...]
    z_ref[...] = z

    @pl.when(i == 0)
    def _():
        s_ref[...] = jnp.zeros_like(s_ref)
        ss_ref[...] = jnp.zeros_like(ss_ref)

    s_ref[...] += jnp.sum(z, axis=0, keepdims=True)
    ss_ref[...] += jnp.sum(z * z, axis=0, keepdims=True)


def _lmid_body(z_ref, m_ref, inv_ref, g_ref, be_ref, w_ref, b_ref,
               zo_ref, s_ref, ss_ref):
    i = pl.program_id(0)
    x = (z_ref[...] - m_ref[...]) * inv_ref[...] * g_ref[...] + be_ref[...]
    x = jnp.maximum(x, 0.0)
    z = jnp.dot(_bf(x), w_ref[...], preferred_element_type=_F32) + b_ref[...]
    zo_ref[...] = z

    @pl.when(i == 0)
    def _():
        s_ref[...] = jnp.zeros_like(s_ref)
        ss_ref[...] = jnp.zeros_like(ss_ref)

    s_ref[...] += jnp.sum(z, axis=0, keepdims=True)
    ss_ref[...] += jnp.sum(z * z, axis=0, keepdims=True)


def _lfin_body(T, H, z_ref, m_ref, inv_ref, g_ref, be_ref, w_ref, b_ref,
               o_ref):
    x = (z_ref[...] - m_ref[...]) * inv_ref[...] * g_ref[...] + be_ref[...]
    x = jnp.maximum(x, 0.0)
    z = jnp.dot(_bf(x), w_ref[...], preferred_element_type=_F32) + b_ref[...]
    o_ref[...] = jnp.max(z.reshape(T // _K, _K, H), axis=1)


def _full_spec(shape):
    return pl.BlockSpec(shape, lambda i: tuple(0 for _ in shape))


def _mlp_max(edges, pos_q, params, F, T):
    (W1, b1, g1, be1), (W2, b2, g2, be2), (W3, b3, _, _) = params
    E, D = edges.shape
    H1, H2, H3 = W1.shape[1], W2.shape[1], W3.shape[1]
    grid = (E // T,)

    z1, s1, ss1 = pl.pallas_call(
        functools.partial(_l1_body, T, F),
        grid=grid,
        in_specs=[
            pl.BlockSpec((T, D), lambda i: (i, 0)),
            pl.BlockSpec((T // _K, 3), lambda i: (i, 0)),
            _full_spec((F + 3, H1)),
            _full_spec((1, H1)),
        ],
        out_specs=(pl.BlockSpec((T, H1), lambda i: (i, 0)),
                   _full_spec((1, H1)), _full_spec((1, H1))),
        out_shape=(jax.ShapeDtypeStruct((E, H1), _F32),
                   jax.ShapeDtypeStruct((1, H1), _F32),
                   jax.ShapeDtypeStruct((1, H1), _F32)),
    )(edges, pos_q, _bf(W1), b1.reshape(1, H1))

    def norm_consts(s, ss):
        m = s / E
        v = ss / E - m * m
        return m, 1.0 / jnp.sqrt(v + 1e-5)

    m1, inv1 = norm_consts(s1, ss1)
    z2, s2, ss2 = pl.pallas_call(
        _lmid_body,
        grid=grid,
        in_specs=[
            pl.BlockSpec((T, H1), lambda i: (i, 0)),
            _full_spec((1, H1)), _full_spec((1, H1)),
            _full_spec((1, H1)), _full_spec((1, H1)),
            _full_spec((H1, H2)), _full_spec((1, H2)),
        ],
        out_specs=(pl.BlockSpec((T, H2), lambda i: (i, 0)),
                   _full_spec((1, H2)), _full_spec((1, H2))),
        out_shape=(jax.ShapeDtypeStruct((E, H2), _F32),
                   jax.ShapeDtypeStruct((1, H2), _F32),
                   jax.ShapeDtypeStruct((1, H2), _F32)),
    )(z1, m1, inv1, g1.reshape(1, H1), be1.reshape(1, H1), _bf(W2),
      b2.reshape(1, H2))

    m2, inv2 = norm_consts(s2, ss2)
    out = pl.pallas_call(
        functools.partial(_lfin_body, T, H3),
        grid=grid,
        in_specs=[
            pl.BlockSpec((T, H2), lambda i: (i, 0)),
            _full_spec((1, H2)), _full_spec((1, H2)),
            _full_spec((1, H2)), _full_spec((1, H2)),
            _full_spec((H2, H3)), _full_spec((1, H3)),
        ],
        out_specs=pl.BlockSpec((T // _K, H3), lambda i: (i, 0)),
        out_shape=jax.ShapeDtypeStruct((E // _K, H3), _F32),
    )(z2, m2, inv2, g2.reshape(1, H2), be2.reshape(1, H2), _bf(W3),
      b3.reshape(1, H3))
    return out


_LEVEL_CFG = {
    16384: (16, 1024, 8192),
    4096: (80, 1024, 8192),
    2048: (144, 512, 8192),
}


def _sa_level(params, x, pos, n_s):
    n = pos.shape[0]
    F = x.shape[1]
    D, CR, T = _LEVEL_CFG[n]
    idx, pos_s = _fps(pos, n_s)
    col = _knn(pos, pos_s)
    table = jnp.concatenate(
        [x, pos, jnp.zeros((n, D - F - 3), _F32)], axis=1)
    edges = _sc_gather(table, col.reshape(-1), CR)
    f = _mlp_max(edges, pos_s, params, F, T)
    return f, pos_s, idx


def kernel(fea, xyz, batch, params1, params2, params3):
    f1, p1, i1 = _sa_level(params1, fea, xyz, 4096)
    f2, p2, i2 = _sa_level(params2, f1, p1, 2048)
    f3, p3, i3 = _sa_level(params3, f2, p2, 1024)
    b3 = batch[i1[i2[i3]]]
    return f3, p3, b3
